# Initial kernel scaffold; baseline (speedup 1.0000x reference)
import jax
import jax.numpy as jnp
from jax.experimental import pallas as pl


def kernel(input, adj, weight, gamma, beta):
    raise NotImplementedError("write your pallas kernel here")


# speedup vs baseline: 1.0173x; 1.0173x over previous
import jax
import jax.numpy as jnp
from jax.experimental import pallas as pl
from jax.experimental.pallas import tpu as pltpu

_SRC = [
    [0, 1, 2, 3, 7, 5, 6, 7],
    [1, 0, 3, 5, 4, 4, 2, 6],
    [2, 3, 3, 1, 6, 7, 4, 5],
    [4, 2, 1, 0, 7, 6, 7, 4],
    [4, 5, 3, 7, 0, 1, 2, 6],
    [5, 4, 7, 6, 1, 5, 5, 2],
    [6, 7, 4, 5, 2, 4, 0, 6],
    [7, 6, 3, 4, 3, 3, 1, 5],
]
_SGN = [
    [1, -1, -1, -1, -1, -1, -1, -1],
    [1, -1, -1, 1, -1, 1, 1, -1],
    [1, 1, 1, -1, -1, -1, 1, 1],
    [1, -1, 1, 1, -1, -1, -1, 1],
    [1, -1, 1, 1, 1, -1, -1, -1],
    [1, -1, 1, -1, 1, 1, 1, -1],
    [1, -1, -1, 1, -1, -1, 1, 1],
    [1, 1, -1, -1, 1, 1, -1, 1],
]


def _pick_tile(n, target):
    t = min(n, target)
    while t > 8 and (n % t or t % 8):
        t -= 8
    return t if n % t == 0 else n


def _main_kernel(x_ref, w_ref, adj_ref, out_ref, stats_ref, ham_s, sup_s):
    i = pl.program_id(0)

    @pl.when(i == 0)
    def _init():
        q = w_ref.shape[0]
        for c in range(8):
            for r in range(8):
                blk = w_ref[:, _SRC[c][r] * q:(_SRC[c][r] + 1) * q]
                ham_s[r * q:(r + 1) * q, c * q:(c + 1) * q] = _SGN[c][r] * blk
        sup_s[...] = jnp.dot(x_ref[...], ham_s[...],
                             preferred_element_type=jnp.float32)
        stats_ref[...] = jnp.zeros_like(stats_ref)

    o = jnp.dot(adj_ref[...], sup_s[...], preferred_element_type=jnp.float32)
    out_ref[...] = o
    stats_ref[0:1, :] += jnp.sum(o, axis=0, keepdims=True)
    stats_ref[1:2, :] += jnp.sum(o * o, axis=0, keepdims=True)


def _bn_kernel(inv_n_ref, y_ref, stats_ref, g_ref, b_ref, out_ref):
    inv_n = inv_n_ref[0]
    mean = stats_ref[0:1, :] * inv_n
    var = stats_ref[1:2, :] * inv_n - mean * mean
    scale = jax.lax.rsqrt(var + 1e-5) * g_ref[0:1, :]
    out_ref[...] = jnp.tanh((y_ref[...] - mean) * scale + b_ref[0:1, :])


def kernel(input, adj, weight, gamma, beta):
    n, f = input.shape
    tm = _pick_tile(n, 400)
    out_unnorm, stats = pl.pallas_call(
        _main_kernel,
        grid=(n // tm,),
        in_specs=[
            pl.BlockSpec((n, f), lambda i: (0, 0)),
            pl.BlockSpec(weight.shape, lambda i: (0, 0)),
            pl.BlockSpec((tm, n), lambda i: (i, 0)),
        ],
        out_specs=[
            pl.BlockSpec((tm, f), lambda i: (i, 0)),
            pl.BlockSpec((8, f), lambda i: (0, 0)),
        ],
        out_shape=[
            jax.ShapeDtypeStruct((n, f), jnp.float32),
            jax.ShapeDtypeStruct((8, f), jnp.float32),
        ],
        scratch_shapes=[
            pltpu.VMEM((f, f), jnp.float32),
            pltpu.VMEM((n, f), jnp.float32),
        ],
        compiler_params=pltpu.CompilerParams(
            dimension_semantics=("arbitrary",)),
    )(input, weight, adj)

    tm2 = _pick_tile(n, 2000)
    inv_n = jnp.full((1,), 1.0 / n, dtype=jnp.float32)
    out = pl.pallas_call(
        _bn_kernel,
        grid=(n // tm2,),
        in_specs=[
            pl.BlockSpec(memory_space=pltpu.SMEM),
            pl.BlockSpec((tm2, f), lambda i: (i, 0)),
            pl.BlockSpec((8, f), lambda i: (0, 0)),
            pl.BlockSpec((1, f), lambda i: (0, 0)),
            pl.BlockSpec((1, f), lambda i: (0, 0)),
        ],
        out_specs=pl.BlockSpec((tm2, f), lambda i: (i, 0)),
        out_shape=jax.ShapeDtypeStruct((n, f), jnp.float32),
    )(inv_n, out_unnorm, stats, gamma.reshape(1, f), beta.reshape(1, f))
    return out


# speedup vs baseline: 1.0673x; 1.0492x over previous
import jax
import jax.numpy as jnp
from jax.experimental import pallas as pl
from jax.experimental.pallas import tpu as pltpu

_SRC = [
    [0, 1, 2, 3, 7, 5, 6, 7],
    [1, 0, 3, 5, 4, 4, 2, 6],
    [2, 3, 3, 1, 6, 7, 4, 5],
    [4, 2, 1, 0, 7, 6, 7, 4],
    [4, 5, 3, 7, 0, 1, 2, 6],
    [5, 4, 7, 6, 1, 5, 5, 2],
    [6, 7, 4, 5, 2, 4, 0, 6],
    [7, 6, 3, 4, 3, 3, 1, 5],
]
_SGN = [
    [1, -1, -1, -1, -1, -1, -1, -1],
    [1, -1, -1, 1, -1, 1, 1, -1],
    [1, 1, 1, -1, -1, -1, 1, 1],
    [1, -1, 1, 1, -1, -1, -1, 1],
    [1, -1, 1, 1, 1, -1, -1, -1],
    [1, -1, 1, -1, 1, 1, 1, -1],
    [1, -1, -1, 1, -1, -1, 1, 1],
    [1, 1, -1, -1, 1, 1, -1, 1],
]


def _pick_tile(n, target):
    t = min(n, target)
    while t > 8 and (n % t or t % 8):
        t -= 8
    return t if n % t == 0 else n


def _make_kernel(n, f, tm):
    tiles = n // tm

    def _kern(x_ref, w_ref, adj_ref, g_ref, b_ref, out_ref,
              ham_s, sup_s, stats_s):
        i = pl.program_id(0)

        @pl.when(i == 0)
        def _init():
            q = w_ref.shape[0]
            for c in range(8):
                for r in range(8):
                    blk = w_ref[:, _SRC[c][r] * q:(_SRC[c][r] + 1) * q]
                    ham_s[r * q:(r + 1) * q, c * q:(c + 1) * q] = \
                        _SGN[c][r] * blk
            sup_s[...] = jnp.dot(x_ref[...], ham_s[...],
                                 preferred_element_type=jnp.float32)
            stats_s[...] = jnp.zeros_like(stats_s)

        o = jnp.dot(adj_ref[...], sup_s[...],
                    preferred_element_type=jnp.float32)
        out_ref[pl.ds(i * tm, tm), :] = o
        stats_s[0:1, :] += jnp.sum(o, axis=0, keepdims=True)
        stats_s[1:2, :] += jnp.sum(o * o, axis=0, keepdims=True)

        @pl.when(i == tiles - 1)
        def _epilogue():
            inv_n = 1.0 / n
            mean = stats_s[0:1, :] * inv_n
            var = stats_s[1:2, :] * inv_n - mean * mean
            scale = jax.lax.rsqrt(var + 1e-5) * g_ref[0:1, :]
            shift = b_ref[0:1, :] - mean * scale
            out_ref[...] = jnp.tanh(out_ref[...] * scale + shift)

    return _kern


def kernel(input, adj, weight, gamma, beta):
    n, f = input.shape
    tm = _pick_tile(n, 400)
    out = pl.pallas_call(
        _make_kernel(n, f, tm),
        grid=(n // tm,),
        in_specs=[
            pl.BlockSpec((n, f), lambda i: (0, 0)),
            pl.BlockSpec(weight.shape, lambda i: (0, 0)),
            pl.BlockSpec((tm, n), lambda i: (i, 0)),
            pl.BlockSpec((1, f), lambda i: (0, 0)),
            pl.BlockSpec((1, f), lambda i: (0, 0)),
        ],
        out_specs=pl.BlockSpec((n, f), lambda i: (0, 0)),
        out_shape=jax.ShapeDtypeStruct((n, f), jnp.float32),
        scratch_shapes=[
            pltpu.VMEM((f, f), jnp.float32),
            pltpu.VMEM((n, f), jnp.float32),
            pltpu.VMEM((8, f), jnp.float32),
        ],
        compiler_params=pltpu.CompilerParams(
            dimension_semantics=("arbitrary",)),
    )(input, weight, adj, gamma.reshape(1, f), beta.reshape(1, f))
    return out


# speedup vs baseline: 1.0696x; 1.0022x over previous
import jax
import jax.numpy as jnp
from jax.experimental import pallas as pl
from jax.experimental.pallas import tpu as pltpu

_SRC = [
    [0, 1, 2, 3, 7, 5, 6, 7],
    [1, 0, 3, 5, 4, 4, 2, 6],
    [2, 3, 3, 1, 6, 7, 4, 5],
    [4, 2, 1, 0, 7, 6, 7, 4],
    [4, 5, 3, 7, 0, 1, 2, 6],
    [5, 4, 7, 6, 1, 5, 5, 2],
    [6, 7, 4, 5, 2, 4, 0, 6],
    [7, 6, 3, 4, 3, 3, 1, 5],
]
_SGN = [
    [1, -1, -1, -1, -1, -1, -1, -1],
    [1, -1, -1, 1, -1, 1, 1, -1],
    [1, 1, 1, -1, -1, -1, 1, 1],
    [1, -1, 1, 1, -1, -1, -1, 1],
    [1, -1, 1, 1, 1, -1, -1, -1],
    [1, -1, 1, -1, 1, 1, 1, -1],
    [1, -1, -1, 1, -1, -1, 1, 1],
    [1, 1, -1, -1, 1, 1, -1, 1],
]


def _pick_tile(n, target):
    t = min(n, target)
    while t > 8 and (n % t or t % 8):
        t -= 8
    return t if n % t == 0 else n


def _make_kernel(n, f, tm):
    tiles = n // tm

    def _kern(x_ref, w_ref, adj_ref, g_ref, b_ref, out_ref,
              ham_s, sup_s, stats_s):
        i = pl.program_id(0)

        @pl.when(i == 0)
        def _init():
            q = w_ref.shape[0]
            for c in range(8):
                for r in range(8):
                    blk = w_ref[:, _SRC[c][r] * q:(_SRC[c][r] + 1) * q]
                    ham_s[r * q:(r + 1) * q, c * q:(c + 1) * q] = \
                        _SGN[c][r] * blk
            sup_s[...] = jnp.dot(x_ref[...], ham_s[...],
                                 preferred_element_type=jnp.float32)
            stats_s[...] = jnp.zeros_like(stats_s)

        o = jnp.dot(adj_ref[...], sup_s[...],
                    precision=jax.lax.Precision.DEFAULT,
                    preferred_element_type=jnp.float32)
        out_ref[pl.ds(i * tm, tm), :] = o
        stats_s[0:1, :] += jnp.sum(o, axis=0, keepdims=True)
        stats_s[1:2, :] += jnp.sum(o * o, axis=0, keepdims=True)

        @pl.when(i == tiles - 1)
        def _epilogue():
            inv_n = 1.0 / n
            mean = stats_s[0:1, :] * inv_n
            var = stats_s[1:2, :] * inv_n - mean * mean
            scale = jax.lax.rsqrt(var + 1e-5) * g_ref[0:1, :]
            shift = b_ref[0:1, :] - mean * scale
            out_ref[...] = jnp.tanh(out_ref[...] * scale + shift)

    return _kern


def kernel(input, adj, weight, gamma, beta):
    n, f = input.shape
    tm = _pick_tile(n, 400)
    out = pl.pallas_call(
        _make_kernel(n, f, tm),
        grid=(n // tm,),
        in_specs=[
            pl.BlockSpec((n, f), lambda i: (0, 0)),
            pl.BlockSpec(weight.shape, lambda i: (0, 0)),
            pl.BlockSpec((tm, n), lambda i: (i, 0)),
            pl.BlockSpec((1, f), lambda i: (0, 0)),
            pl.BlockSpec((1, f), lambda i: (0, 0)),
        ],
        out_specs=pl.BlockSpec((n, f), lambda i: (0, 0)),
        out_shape=jax.ShapeDtypeStruct((n, f), jnp.float32),
        scratch_shapes=[
            pltpu.VMEM((f, f), jnp.float32),
            pltpu.VMEM((n, f), jnp.float32),
            pltpu.VMEM((8, f), jnp.float32),
        ],
        compiler_params=pltpu.CompilerParams(
            dimension_semantics=("arbitrary",)),
    )(input, weight, adj, gamma.reshape(1, f), beta.reshape(1, f))
    return out
